# baseline (speedup 1.0000x reference)
import jax
import jax.numpy as jnp
from jax import lax
from jax.experimental import pallas as pl
from jax.experimental.pallas import tpu as pltpu
from jax.experimental.pallas import tpu_sc as plsc
import functools

_N_ATOMS = 10000
_N_FRAGS = 2000
_N_EDGES = 320000
_D = 128
_B = 100
_APG = 100
_FPG = 20

_NC = 2
_NS = 16
_APAD = 10240
_HALF = _APAD // _NC
_TPAD = _HALF + 8
_ZROW = _HALF
_EPT = 20480
_EPAD = _NS * _EPT
_SEGE = 2048
_NSEG = _EPT // _SEGE
_CH = 128
_TRASH = _SEGE + 256
_LCAP = _TRASH + 16
_SROWS = _HALF // _NS


def _sc_body(src_hbm, dst_hbm, xa_hbm, out_hbm,
             src_seg, dst_seg, list_s, list_d, buf0, buf1, zbuf,
             table, acc, gsem0, gsem1, ssem0, ssem1):
    cid = lax.axis_index("c")
    sid = lax.axis_index("s")
    dlo = cid * _HALF

    def _zrow(r, c):
        for c8 in range(8):
            buf0[r, pl.ds(c8 * 16, 16)] = jnp.zeros((16,), jnp.float32)
        return c
    lax.fori_loop(0, _CH, _zrow, 0)

    def _zrow2(r, c):
        for c8 in range(8):
            zbuf[r, pl.ds(c8 * 16, 16)] = jnp.zeros((16,), jnp.float32)
        return c
    lax.fori_loop(0, 8, _zrow2, 0)

    abase = sid * _SROWS
    pltpu.sync_copy(buf0, acc.at[pl.ds(abase, _CH)])
    pltpu.sync_copy(buf0, acc.at[pl.ds(abase + _CH, _CH)])
    pltpu.sync_copy(buf0.at[pl.ds(0, 64)], acc.at[pl.ds(abase + 2 * _CH, 64)])

    def stage_table(half_lo):
        pltpu.sync_copy(xa_hbm.at[pl.ds(half_lo + sid * _SROWS, _SROWS)],
                        table.at[pl.ds(sid * _SROWS, _SROWS)])

        @pl.when(sid == _NS - 1)
        def _():
            pltpu.sync_copy(zbuf, table.at[pl.ds(_HALF, 8)])

    def start_gather(i, buf, sem):
        pltpu.make_async_copy(table.at[list_s.at[i]], buf, sem).start()

    def wait_gather(buf, sem):
        pltpu.make_async_copy(table.at[list_s.at[0]], buf, sem).wait()

    def start_scatter(i, buf, sem):
        pltpu.async_copy(buf, acc.at[list_d.at[i]], sem, add=True)

    def wait_scatter(buf, sem):
        pltpu.make_async_copy(buf, acc.at[list_d.at[0]], sem).wait()

    def run_pass(slo):
        slo_v = jnp.full((16,), slo, jnp.int32)
        half_v = jnp.full((16,), _HALF, jnp.int32)
        shi_v = slo_v + half_v
        dlo_v = jnp.full((16,), dlo, jnp.int32)
        dhi_v = dlo_v + half_v
        trash_v = jnp.full((16,), _TRASH, jnp.int32) + lax.iota(jnp.int32, 16)
        one_v = jnp.full((16,), 1, jnp.int32)
        c7_v = jnp.full((16,), 7, jnp.int32)
        c127_v = jnp.full((16,), 127, jnp.int32)
        iota_v = lax.iota(jnp.int32, 16)

        def _segment(g, c):
            ebase = g * _SEGE
            pltpu.sync_copy(src_hbm.at[sid, pl.ds(ebase, _SEGE)], src_seg)
            pltpu.sync_copy(dst_hbm.at[sid, pl.ds(ebase, _SEGE)], dst_seg)

            def _cvec(v, off):
                sv = src_seg[pl.ds(v * 16, 16)]
                dv = dst_seg[pl.ds(v * 16, 16)]
                keep = ((sv >= slo_v) & (sv < shi_v) &
                        (dv >= dlo_v) & (dv < dhi_v))
                ki = keep.astype(jnp.int32)
                cum = plsc.cumsum(ki)
                off_v = jnp.full((16,), off, jnp.int32)
                pos = (off_v + cum - one_v) * ki + trash_v * (one_v - ki)
                prow = lax.shift_right_logical(pos, c7_v)
                pcol = pos & c127_v
                plsc.store_scatter(list_s, [prow, pcol], sv - slo_v)
                plsc.store_scatter(list_d, [prow, pcol], dv - dlo_v)
                return off + jnp.sum(ki)
            off = lax.fori_loop(0, _SEGE // 16, _cvec, jnp.int32(0))

            zsv = jnp.full((16,), _ZROW, jnp.int32)
            zdv = jnp.zeros((16,), jnp.int32)
            off_v2 = jnp.full((16,), off, jnp.int32)
            for t in range(16):
                ppos = off_v2 + jnp.full((16,), t * 16, jnp.int32) + iota_v
                prow = lax.shift_right_logical(ppos, c7_v)
                pcol = ppos & c127_v
                plsc.store_scatter(list_s, [prow, pcol], zsv)
                plsc.store_scatter(list_d, [prow, pcol], zdv)
            nprs = (off + 255) // 256

            @pl.when(nprs > 0)
            def _():
                start_gather(0, buf0, gsem0)

            def _pair(j, c2):
                i0 = 2 * j
                start_gather(i0 + 1, buf1, gsem1)
                wait_gather(buf0, gsem0)
                start_scatter(i0, buf0, ssem0)
                wait_gather(buf1, gsem1)
                start_scatter(i0 + 1, buf1, ssem1)
                wait_scatter(buf0, ssem0)

                @pl.when(j < nprs - 1)
                def _():
                    start_gather(i0 + 2, buf0, gsem0)

                wait_scatter(buf1, ssem1)
                return c2
            lax.fori_loop(0, nprs, _pair, 0)
            return c
        lax.fori_loop(0, _NSEG, _segment, 0)

    stage_table(cid * _HALF)
    plsc.subcore_barrier()
    run_pass(dlo)
    plsc.subcore_barrier()
    stage_table((1 - cid) * _HALF)
    plsc.subcore_barrier()
    run_pass((1 - cid) * _HALF)
    plsc.subcore_barrier()

    pltpu.sync_copy(acc.at[pl.ds(abase, _SROWS)],
                    out_hbm.at[pl.ds(dlo + abase, _SROWS)])


@functools.cache
def _sc_scatter():
    return pl.kernel(
        _sc_body,
        out_type=jax.ShapeDtypeStruct((_APAD, _D), jnp.float32),
        mesh=plsc.VectorSubcoreMesh(core_axis_name="c", subcore_axis_name="s"),
        scratch_types=[
            pltpu.VMEM((_SEGE,), jnp.int32),
            pltpu.VMEM((_SEGE,), jnp.int32),
            pltpu.VMEM((19, _CH), jnp.int32),
            pltpu.VMEM((19, _CH), jnp.int32),
            pltpu.VMEM((_CH, _D), jnp.float32),
            pltpu.VMEM((_CH, _D), jnp.float32),
            pltpu.VMEM((8, _D), jnp.float32),
            pltpu.VMEM_SHARED((_TPAD, _D), jnp.float32),
            pltpu.VMEM_SHARED((_HALF, _D), jnp.float32),
            pltpu.SemaphoreType.DMA,
            pltpu.SemaphoreType.DMA,
            pltpu.SemaphoreType.DMA,
            pltpu.SemaphoreType.DMA,
        ],
        compiler_params=pltpu.CompilerParams(needs_layout_passes=False),
        name="edge_scatter_add_sc",
    )


_GPB = 10
_AROWS = _GPB * _APG
_FROWS = _GPB * _FPG


def _tc_body(xa, g, xf, wm, ws, wf, out):
    h = jnp.dot(xa[...], ws[...], preferred_element_type=jnp.float32)
    h = h + jnp.dot(g[...], wm[...], preferred_element_type=jnp.float32)
    h = jnp.maximum(h, 0.0)
    f = jnp.maximum(
        jnp.dot(xf[...], wf[...], preferred_element_type=jnp.float32), 0.0)
    hs = jnp.sum(h.reshape(_GPB, _APG, _D), axis=1)
    fs = jnp.sum(f.reshape(_GPB, _FPG, _D), axis=1)
    out[0] = (hs + fs) * (1.0 / (_APG + _FPG))


@functools.partial(jax.jit)
def _tc_finish(x_atom, gp, x_frag, W_msg, W_self, W_frag):
    nb = _B // _GPB
    return pl.pallas_call(
        _tc_body,
        grid=(nb,),
        in_specs=[
            pl.BlockSpec((_AROWS, _D), lambda b: (b, 0)),
            pl.BlockSpec((_AROWS, _D), lambda b: (b, 0)),
            pl.BlockSpec((_FROWS, _D), lambda b: (b, 0)),
            pl.BlockSpec((_D, _D), lambda b: (0, 0)),
            pl.BlockSpec((_D, _D), lambda b: (0, 0)),
            pl.BlockSpec((_D, _D), lambda b: (0, 0)),
        ],
        out_specs=pl.BlockSpec((1, _GPB, _D), lambda b: (b, 0, 0)),
        out_shape=jax.ShapeDtypeStruct((nb, _GPB, _D), jnp.float32),
        name="embed_pool_tc",
    )(x_atom, gp, x_frag, W_msg, W_self, W_frag).reshape(_B, _D)


def kernel(x_atom, x_frag, edge_index, W_msg, W_self, W_frag):
    ei = edge_index.astype(jnp.int32)
    npad = _EPAD - _N_EDGES
    pad_src = jnp.zeros((npad,), jnp.int32)
    pad_dst = _N_ATOMS + (jnp.arange(npad, dtype=jnp.int32)
                          % (_APAD - _N_ATOMS))
    src = jnp.concatenate([ei[0], pad_src]).reshape(_NS, _EPT)
    dst = jnp.concatenate([ei[1], pad_dst]).reshape(_NS, _EPT)
    xa_pad = jnp.concatenate(
        [x_atom, jnp.zeros((_APAD - _N_ATOMS, _D), jnp.float32)])
    gp = _sc_scatter()(src, dst, xa_pad)
    return _tc_finish(x_atom, gp[:_N_ATOMS], x_frag, W_msg, W_self, W_frag)


# speedup vs baseline: 1.2020x; 1.2020x over previous
import jax
import jax.numpy as jnp
from jax import lax
from jax.experimental import pallas as pl
from jax.experimental.pallas import tpu as pltpu
from jax.experimental.pallas import tpu_sc as plsc
import functools

_N_ATOMS = 10000
_N_FRAGS = 2000
_N_EDGES = 320000
_D = 128
_B = 100
_APG = 100
_FPG = 20

_NC = 2
_NS = 16
_APAD = 10240
_HALF = _APAD // _NC
_TPAD = _HALF + 8
_ZROW = _HALF
_EPT = 20480
_EPAD = _NS * _EPT
_SEGE = 2048
_NSEG = _EPT // _SEGE
_CH = 128
_TRASH = _SEGE + 256
_LCAP = _TRASH + 16
_SROWS = _HALF // _NS


def _sc_body(src_hbm, dst_hbm, xa_hbm, out_hbm,
             src_seg, dst_seg, list_s, list_d, buf0, buf1, zbuf,
             table, acc, gsem0, gsem1, ssem0, ssem1):
    cid = lax.axis_index("c")
    sid = lax.axis_index("s")
    dlo = cid * _HALF

    def _zrow(r, c):
        for c8 in range(8):
            buf0[r, pl.ds(c8 * 16, 16)] = jnp.zeros((16,), jnp.float32)
        return c
    lax.fori_loop(0, _CH, _zrow, 0)

    def _zrow2(r, c):
        for c8 in range(8):
            zbuf[r, pl.ds(c8 * 16, 16)] = jnp.zeros((16,), jnp.float32)
        return c
    lax.fori_loop(0, 8, _zrow2, 0)

    abase = sid * _SROWS
    pltpu.sync_copy(buf0, acc.at[pl.ds(abase, _CH)])
    pltpu.sync_copy(buf0, acc.at[pl.ds(abase + _CH, _CH)])
    pltpu.sync_copy(buf0.at[pl.ds(0, 64)], acc.at[pl.ds(abase + 2 * _CH, 64)])

    def stage_table(half_lo):
        pltpu.sync_copy(xa_hbm.at[pl.ds(half_lo + sid * _SROWS, _SROWS)],
                        table.at[pl.ds(sid * _SROWS, _SROWS)])

        @pl.when(sid == _NS - 1)
        def _():
            pltpu.sync_copy(zbuf, table.at[pl.ds(_HALF, 8)])

    def start_gather(i, buf, sem):
        pltpu.make_async_copy(table.at[list_s.at[i]], buf, sem).start()

    def wait_gather(buf, sem):
        pltpu.make_async_copy(table.at[list_s.at[0]], buf, sem).wait()

    def start_scatter(i, buf, sem):
        pltpu.async_copy(buf, acc.at[list_d.at[i]], sem, add=True)

    def wait_scatter(buf, sem):
        pltpu.make_async_copy(buf, acc.at[list_d.at[0]], sem).wait()

    def run_pass(slo):
        slo_v = jnp.full((16,), slo, jnp.int32)
        half_v = jnp.full((16,), _HALF, jnp.int32)
        shi_v = slo_v + half_v
        dlo_v = jnp.full((16,), dlo, jnp.int32)
        dhi_v = dlo_v + half_v
        trash_v = jnp.full((16,), _TRASH, jnp.int32) + lax.iota(jnp.int32, 16)
        one_v = jnp.full((16,), 1, jnp.int32)
        c7_v = jnp.full((16,), 7, jnp.int32)
        c127_v = jnp.full((16,), 127, jnp.int32)
        iota_v = lax.iota(jnp.int32, 16)

        def _segment(g, c):
            ebase = g * _SEGE
            pltpu.sync_copy(src_hbm.at[sid, pl.ds(ebase, _SEGE)], src_seg)
            pltpu.sync_copy(dst_hbm.at[sid, pl.ds(ebase, _SEGE)], dst_seg)

            def _cvec(v, off):
                sv = src_seg[pl.ds(v * 16, 16)]
                dv = dst_seg[pl.ds(v * 16, 16)]
                keep = ((sv >= slo_v) & (sv < shi_v) &
                        (dv >= dlo_v) & (dv < dhi_v))
                ki = keep.astype(jnp.int32)
                cum = plsc.cumsum(ki)
                off_v = jnp.full((16,), off, jnp.int32)
                pos = (off_v + cum - one_v) * ki + trash_v * (one_v - ki)
                prow = lax.shift_right_logical(pos, c7_v)
                pcol = pos & c127_v
                plsc.store_scatter(list_s, [prow, pcol], sv - slo_v)
                plsc.store_scatter(list_d, [prow, pcol], dv - dlo_v)
                return off + jnp.sum(ki)
            off = lax.fori_loop(0, _SEGE // 16, _cvec, jnp.int32(0))

            zsv = jnp.full((16,), _ZROW, jnp.int32)
            zdv = jnp.zeros((16,), jnp.int32)
            off_v2 = jnp.full((16,), off, jnp.int32)
            for t in range(8):
                ppos = off_v2 + jnp.full((16,), t * 16, jnp.int32) + iota_v
                prow = lax.shift_right_logical(ppos, c7_v)
                pcol = ppos & c127_v
                plsc.store_scatter(list_s, [prow, pcol], zsv)
                plsc.store_scatter(list_d, [prow, pcol], zdv)
            nch = (off + 127) // 128
            nprs = nch // 2
            odd = (nch % 2) == 1

            @pl.when(nch > 0)
            def _():
                start_gather(0, buf0, gsem0)

            def _pair(j, c2):
                i0 = 2 * j
                start_gather(i0 + 1, buf1, gsem1)
                wait_gather(buf0, gsem0)
                pltpu.sync_copy(buf0, acc.at[list_d.at[i0]], add=True)

                @pl.when((j < nprs - 1) | odd)
                def _():
                    start_gather(i0 + 2, buf0, gsem0)

                wait_gather(buf1, gsem1)
                pltpu.sync_copy(buf1, acc.at[list_d.at[i0 + 1]], add=True)
                return c2
            lax.fori_loop(0, nprs, _pair, 0)

            @pl.when(odd)
            def _():
                wait_gather(buf0, gsem0)
                pltpu.sync_copy(buf0, acc.at[list_d.at[2 * nprs]], add=True)
            return c
        lax.fori_loop(0, _NSEG, _segment, 0)

    stage_table(cid * _HALF)
    plsc.subcore_barrier()
    run_pass(dlo)
    plsc.subcore_barrier()
    stage_table((1 - cid) * _HALF)
    plsc.subcore_barrier()
    run_pass((1 - cid) * _HALF)
    plsc.subcore_barrier()

    pltpu.sync_copy(acc.at[pl.ds(abase, _SROWS)],
                    out_hbm.at[pl.ds(dlo + abase, _SROWS)])


@functools.cache
def _sc_scatter():
    return pl.kernel(
        _sc_body,
        out_type=jax.ShapeDtypeStruct((_APAD, _D), jnp.float32),
        mesh=plsc.VectorSubcoreMesh(core_axis_name="c", subcore_axis_name="s"),
        scratch_types=[
            pltpu.VMEM((_SEGE,), jnp.int32),
            pltpu.VMEM((_SEGE,), jnp.int32),
            pltpu.VMEM((19, _CH), jnp.int32),
            pltpu.VMEM((19, _CH), jnp.int32),
            pltpu.VMEM((_CH, _D), jnp.float32),
            pltpu.VMEM((_CH, _D), jnp.float32),
            pltpu.VMEM((8, _D), jnp.float32),
            pltpu.VMEM_SHARED((_TPAD, _D), jnp.float32),
            pltpu.VMEM_SHARED((_HALF, _D), jnp.float32),
            pltpu.SemaphoreType.DMA,
            pltpu.SemaphoreType.DMA,
            pltpu.SemaphoreType.DMA,
            pltpu.SemaphoreType.DMA,
        ],
        compiler_params=pltpu.CompilerParams(needs_layout_passes=False),
        name="edge_scatter_add_sc",
    )


_GPB = 10
_AROWS = _GPB * _APG
_FROWS = _GPB * _FPG


def _tc_body(xa, g, xf, wm, ws, wf, out):
    h = jnp.dot(xa[...], ws[...], preferred_element_type=jnp.float32)
    h = h + jnp.dot(g[...], wm[...], preferred_element_type=jnp.float32)
    h = jnp.maximum(h, 0.0)
    f = jnp.maximum(
        jnp.dot(xf[...], wf[...], preferred_element_type=jnp.float32), 0.0)
    hs = jnp.sum(h.reshape(_GPB, _APG, _D), axis=1)
    fs = jnp.sum(f.reshape(_GPB, _FPG, _D), axis=1)
    out[0] = (hs + fs) * (1.0 / (_APG + _FPG))


@functools.partial(jax.jit)
def _tc_finish(x_atom, gp, x_frag, W_msg, W_self, W_frag):
    nb = _B // _GPB
    return pl.pallas_call(
        _tc_body,
        grid=(nb,),
        in_specs=[
            pl.BlockSpec((_AROWS, _D), lambda b: (b, 0)),
            pl.BlockSpec((_AROWS, _D), lambda b: (b, 0)),
            pl.BlockSpec((_FROWS, _D), lambda b: (b, 0)),
            pl.BlockSpec((_D, _D), lambda b: (0, 0)),
            pl.BlockSpec((_D, _D), lambda b: (0, 0)),
            pl.BlockSpec((_D, _D), lambda b: (0, 0)),
        ],
        out_specs=pl.BlockSpec((1, _GPB, _D), lambda b: (b, 0, 0)),
        out_shape=jax.ShapeDtypeStruct((nb, _GPB, _D), jnp.float32),
        name="embed_pool_tc",
    )(x_atom, gp, x_frag, W_msg, W_self, W_frag).reshape(_B, _D)


def kernel(x_atom, x_frag, edge_index, W_msg, W_self, W_frag):
    ei = edge_index.astype(jnp.int32)
    npad = _EPAD - _N_EDGES
    pad_src = jnp.zeros((npad,), jnp.int32)
    pad_dst = _N_ATOMS + (jnp.arange(npad, dtype=jnp.int32)
                          % (_APAD - _N_ATOMS))
    src = jnp.concatenate([ei[0], pad_src]).reshape(_NS, _EPT)
    dst = jnp.concatenate([ei[1], pad_dst]).reshape(_NS, _EPT)
    xa_pad = jnp.concatenate(
        [x_atom, jnp.zeros((_APAD - _N_ATOMS, _D), jnp.float32)])
    gp = _sc_scatter()(src, dst, xa_pad)
    return _tc_finish(x_atom, gp, x_frag, W_msg, W_self, W_frag)


# speedup vs baseline: 1.2930x; 1.0757x over previous
import jax
import jax.numpy as jnp
from jax import lax
from jax.experimental import pallas as pl
from jax.experimental.pallas import tpu as pltpu
from jax.experimental.pallas import tpu_sc as plsc
import functools

_N_ATOMS = 10000
_N_FRAGS = 2000
_N_EDGES = 320000
_D = 128
_B = 100
_APG = 100
_FPG = 20

_NC = 2
_NS = 16
_APAD = 10240
_HALF = _APAD // _NC
_TPAD = _HALF + 8
_ZROW = _HALF
_EPT = 20480
_EPAD = _NS * _EPT
_SEGE = 2048
_NSEG = _EPT // _SEGE
_CH = 128
_TRASH = _SEGE + 256
_LCAP = _TRASH + 16
_SROWS = _HALF // _NS


def _sc_body(src_hbm, dst_hbm, xa_hbm, out_hbm,
             src_seg, dst_seg, src_seg1, dst_seg1, list_s, list_d,
             buf0, buf1, zbuf, table, acc, gsem0, gsem1, ssem0, ssem1):
    cid = lax.axis_index("c")
    sid = lax.axis_index("s")
    dlo = cid * _HALF

    def _zrow(r, c):
        for c8 in range(8):
            buf0[r, pl.ds(c8 * 16, 16)] = jnp.zeros((16,), jnp.float32)
        return c
    lax.fori_loop(0, _CH, _zrow, 0)

    def _zrow2(r, c):
        for c8 in range(8):
            zbuf[r, pl.ds(c8 * 16, 16)] = jnp.zeros((16,), jnp.float32)
        return c
    lax.fori_loop(0, 8, _zrow2, 0)

    abase = sid * _SROWS
    pltpu.sync_copy(buf0, acc.at[pl.ds(abase, _CH)])
    pltpu.sync_copy(buf0, acc.at[pl.ds(abase + _CH, _CH)])
    pltpu.sync_copy(buf0.at[pl.ds(0, 64)], acc.at[pl.ds(abase + 2 * _CH, 64)])

    def stage_table(half_lo):
        pltpu.sync_copy(xa_hbm.at[pl.ds(half_lo + sid * _SROWS, _SROWS)],
                        table.at[pl.ds(sid * _SROWS, _SROWS)])

        @pl.when(sid == _NS - 1)
        def _():
            pltpu.sync_copy(zbuf, table.at[pl.ds(_HALF, 8)])

    def start_gather(i, buf, sem):
        pltpu.make_async_copy(table.at[list_s.at[i]], buf, sem).start()

    def wait_gather(buf, sem):
        pltpu.make_async_copy(table.at[list_s.at[0]], buf, sem).wait()

    def start_scatter(i, buf, sem):
        pltpu.async_copy(buf, acc.at[list_d.at[i]], sem, add=True)

    def wait_scatter(buf, sem):
        pltpu.make_async_copy(buf, acc.at[list_d.at[0]], sem).wait()

    def run_pass(slo):
        slo_v = jnp.full((16,), slo, jnp.int32)
        half_v = jnp.full((16,), _HALF, jnp.int32)
        shi_v = slo_v + half_v
        dlo_v = jnp.full((16,), dlo, jnp.int32)
        dhi_v = dlo_v + half_v
        trash_v = jnp.full((16,), _TRASH, jnp.int32) + lax.iota(jnp.int32, 16)
        one_v = jnp.full((16,), 1, jnp.int32)
        c7_v = jnp.full((16,), 7, jnp.int32)
        c127_v = jnp.full((16,), 127, jnp.int32)
        iota_v = lax.iota(jnp.int32, 16)

        def start_seg(g, sbuf, dbuf, sem):
            ebase = g * _SEGE
            pltpu.make_async_copy(
                src_hbm.at[sid, pl.ds(ebase, _SEGE)], sbuf, sem).start()
            pltpu.make_async_copy(
                dst_hbm.at[sid, pl.ds(ebase, _SEGE)], dbuf, sem).start()

        def wait_seg(sbuf, dbuf, sem):
            pltpu.make_async_copy(
                src_hbm.at[sid, pl.ds(0, _SEGE)], sbuf, sem).wait()
            pltpu.make_async_copy(
                dst_hbm.at[sid, pl.ds(0, _SEGE)], dbuf, sem).wait()

        def _do_segment(sbuf, dbuf):
            def _cvec(v, off):
                sv = sbuf[pl.ds(v * 16, 16)]
                dv = dbuf[pl.ds(v * 16, 16)]
                keep = ((sv >= slo_v) & (sv < shi_v) &
                        (dv >= dlo_v) & (dv < dhi_v))
                ki = keep.astype(jnp.int32)
                cum = plsc.cumsum(ki)
                off_v = jnp.full((16,), off, jnp.int32)
                pos = (off_v + cum - one_v) * ki + trash_v * (one_v - ki)
                prow = lax.shift_right_logical(pos, c7_v)
                pcol = pos & c127_v
                plsc.store_scatter(list_s, [prow, pcol], sv - slo_v)
                plsc.store_scatter(list_d, [prow, pcol], dv - dlo_v)
                return off + jnp.sum(ki)
            off = lax.fori_loop(0, _SEGE // 16, _cvec, jnp.int32(0))

            zsv = jnp.full((16,), _ZROW, jnp.int32)
            zdv = jnp.zeros((16,), jnp.int32)
            off_v2 = jnp.full((16,), off, jnp.int32)
            for t in range(8):
                ppos = off_v2 + jnp.full((16,), t * 16, jnp.int32) + iota_v
                prow = lax.shift_right_logical(ppos, c7_v)
                pcol = ppos & c127_v
                plsc.store_scatter(list_s, [prow, pcol], zsv)
                plsc.store_scatter(list_d, [prow, pcol], zdv)
            nch = (off + 127) // 128
            nprs = nch // 2
            odd = (nch % 2) == 1

            @pl.when(nch > 0)
            def _():
                start_gather(0, buf0, gsem0)

            def _pair(j, c2):
                i0 = 2 * j
                start_gather(i0 + 1, buf1, gsem1)
                wait_gather(buf0, gsem0)
                pltpu.sync_copy(buf0, acc.at[list_d.at[i0]], add=True)

                @pl.when((j < nprs - 1) | odd)
                def _():
                    start_gather(i0 + 2, buf0, gsem0)

                wait_gather(buf1, gsem1)
                pltpu.sync_copy(buf1, acc.at[list_d.at[i0 + 1]], add=True)
                return c2
            lax.fori_loop(0, nprs, _pair, 0)

            @pl.when(odd)
            def _():
                wait_gather(buf0, gsem0)
                pltpu.sync_copy(buf0, acc.at[list_d.at[2 * nprs]], add=True)

        start_seg(0, src_seg, dst_seg, ssem0)

        def _segpair(g2, c):
            g0 = 2 * g2
            wait_seg(src_seg, dst_seg, ssem0)
            start_seg(g0 + 1, src_seg1, dst_seg1, ssem1)
            _do_segment(src_seg, dst_seg)
            wait_seg(src_seg1, dst_seg1, ssem1)

            @pl.when(g2 < _NSEG // 2 - 1)
            def _():
                start_seg(g0 + 2, src_seg, dst_seg, ssem0)

            _do_segment(src_seg1, dst_seg1)
            return c
        lax.fori_loop(0, _NSEG // 2, _segpair, 0)

    stage_table(cid * _HALF)
    plsc.subcore_barrier()
    run_pass(dlo)
    plsc.subcore_barrier()
    stage_table((1 - cid) * _HALF)
    plsc.subcore_barrier()
    run_pass((1 - cid) * _HALF)
    plsc.subcore_barrier()

    pltpu.sync_copy(acc.at[pl.ds(abase, _SROWS)],
                    out_hbm.at[pl.ds(dlo + abase, _SROWS)])


@functools.cache
def _sc_scatter():
    return pl.kernel(
        _sc_body,
        out_type=jax.ShapeDtypeStruct((_APAD, _D), jnp.float32),
        mesh=plsc.VectorSubcoreMesh(core_axis_name="c", subcore_axis_name="s"),
        scratch_types=[
            pltpu.VMEM((_SEGE,), jnp.int32),
            pltpu.VMEM((_SEGE,), jnp.int32),
            pltpu.VMEM((_SEGE,), jnp.int32),
            pltpu.VMEM((_SEGE,), jnp.int32),
            pltpu.VMEM((19, _CH), jnp.int32),
            pltpu.VMEM((19, _CH), jnp.int32),
            pltpu.VMEM((_CH, _D), jnp.float32),
            pltpu.VMEM((_CH, _D), jnp.float32),
            pltpu.VMEM((8, _D), jnp.float32),
            pltpu.VMEM_SHARED((_TPAD, _D), jnp.float32),
            pltpu.VMEM_SHARED((_HALF, _D), jnp.float32),
            pltpu.SemaphoreType.DMA,
            pltpu.SemaphoreType.DMA,
            pltpu.SemaphoreType.DMA,
            pltpu.SemaphoreType.DMA,
        ],
        compiler_params=pltpu.CompilerParams(needs_layout_passes=False),
        name="edge_scatter_add_sc",
    )


_GPB = 10
_AROWS = _GPB * _APG
_FROWS = _GPB * _FPG


def _tc_body(xa, g, xf, wm, ws, wf, out):
    h = jnp.dot(xa[...], ws[...], preferred_element_type=jnp.float32)
    h = h + jnp.dot(g[...], wm[...], preferred_element_type=jnp.float32)
    h = jnp.maximum(h, 0.0)
    f = jnp.maximum(
        jnp.dot(xf[...], wf[...], preferred_element_type=jnp.float32), 0.0)
    hs = jnp.sum(h.reshape(_GPB, _APG, _D), axis=1)
    fs = jnp.sum(f.reshape(_GPB, _FPG, _D), axis=1)
    out[0] = (hs + fs) * (1.0 / (_APG + _FPG))


@functools.partial(jax.jit)
def _tc_finish(x_atom, gp, x_frag, W_msg, W_self, W_frag):
    nb = _B // _GPB
    return pl.pallas_call(
        _tc_body,
        grid=(nb,),
        in_specs=[
            pl.BlockSpec((_AROWS, _D), lambda b: (b, 0)),
            pl.BlockSpec((_AROWS, _D), lambda b: (b, 0)),
            pl.BlockSpec((_FROWS, _D), lambda b: (b, 0)),
            pl.BlockSpec((_D, _D), lambda b: (0, 0)),
            pl.BlockSpec((_D, _D), lambda b: (0, 0)),
            pl.BlockSpec((_D, _D), lambda b: (0, 0)),
        ],
        out_specs=pl.BlockSpec((1, _GPB, _D), lambda b: (b, 0, 0)),
        out_shape=jax.ShapeDtypeStruct((nb, _GPB, _D), jnp.float32),
        name="embed_pool_tc",
    )(x_atom, gp, x_frag, W_msg, W_self, W_frag).reshape(_B, _D)


def kernel(x_atom, x_frag, edge_index, W_msg, W_self, W_frag):
    ei = edge_index.astype(jnp.int32)
    npad = _EPAD - _N_EDGES
    pad_src = jnp.zeros((npad,), jnp.int32)
    pad_dst = _N_ATOMS + (jnp.arange(npad, dtype=jnp.int32)
                          % (_APAD - _N_ATOMS))
    src = jnp.concatenate([ei[0], pad_src]).reshape(_NS, _EPT)
    dst = jnp.concatenate([ei[1], pad_dst]).reshape(_NS, _EPT)
    xa_pad = jnp.concatenate(
        [x_atom, jnp.zeros((_APAD - _N_ATOMS, _D), jnp.float32)])
    gp = _sc_scatter()(src, dst, xa_pad)
    return _tc_finish(x_atom, gp, x_frag, W_msg, W_self, W_frag)


# speedup vs baseline: 1.4494x; 1.1209x over previous
import jax
import jax.numpy as jnp
from jax import lax
from jax.experimental import pallas as pl
from jax.experimental.pallas import tpu as pltpu
from jax.experimental.pallas import tpu_sc as plsc
import functools

_N_ATOMS = 10000
_N_FRAGS = 2000
_N_EDGES = 320000
_D = 128
_B = 100
_APG = 100
_FPG = 20

_NC = 2
_NS = 16
_APAD = 10240
_HALF = _APAD // _NC
_TPAD = _HALF + 8
_ZROW = _HALF
_EPT = _N_EDGES // _NS
_SEGE = 2000
_NSEG = _EPT // _SEGE
_CH = 128
_TRASH = 17 * _CH
_LROWS = 18
_SROWS = _HALF // _NS


def _sc_body(src_hbm, dst_hbm, xa_hbm, out_hbm,
             src_seg, dst_seg, src_seg1, dst_seg1, list_s, list_d,
             buf0, buf1, zbuf, table, acc, gsem0, gsem1, ssem0, ssem1):
    cid = lax.axis_index("c")
    sid = lax.axis_index("s")
    dlo = cid * _HALF

    def _zrow(r, c):
        for c8 in range(8):
            buf0[r, pl.ds(c8 * 16, 16)] = jnp.zeros((16,), jnp.float32)
        return c
    lax.fori_loop(0, _CH, _zrow, 0)

    def _zrow2(r, c):
        for c8 in range(8):
            zbuf[r, pl.ds(c8 * 16, 16)] = jnp.zeros((16,), jnp.float32)
        return c
    lax.fori_loop(0, 8, _zrow2, 0)

    abase = sid * _SROWS
    pltpu.sync_copy(buf0, acc.at[pl.ds(abase, _CH)])
    pltpu.sync_copy(buf0, acc.at[pl.ds(abase + _CH, _CH)])
    pltpu.sync_copy(buf0.at[pl.ds(0, 64)], acc.at[pl.ds(abase + 2 * _CH, 64)])

    def stage_table(half_lo):
        rs = half_lo + sid * _SROWS

        @pl.when(rs + _SROWS <= _N_ATOMS)
        def _():
            pltpu.sync_copy(xa_hbm.at[pl.ds(rs, _SROWS)],
                            table.at[pl.ds(sid * _SROWS, _SROWS)])

        @pl.when(rs + _SROWS > _N_ATOMS)
        def _():
            pltpu.sync_copy(xa_hbm.at[pl.ds(rs, _N_ATOMS - 9920)],
                            table.at[pl.ds(sid * _SROWS, _N_ATOMS - 9920)])

        @pl.when(sid == _NS - 1)
        def _():
            pltpu.sync_copy(zbuf, table.at[pl.ds(_HALF, 8)])

    def start_gather(i, buf, sem):
        pltpu.make_async_copy(table.at[list_s.at[i]], buf, sem).start()

    def wait_gather(buf, sem):
        pltpu.make_async_copy(table.at[list_s.at[0]], buf, sem).wait()

    def start_scatter(i, buf, sem):
        pltpu.async_copy(buf, acc.at[list_d.at[i]], sem, add=True)

    def wait_scatter(buf, sem):
        pltpu.make_async_copy(buf, acc.at[list_d.at[0]], sem).wait()

    def run_pass(slo):
        slo_v = jnp.full((16,), slo, jnp.int32)
        half_v = jnp.full((16,), _HALF, jnp.int32)
        shi_v = slo_v + half_v
        dlo_v = jnp.full((16,), dlo, jnp.int32)
        dhi_v = dlo_v + half_v
        trash_v = jnp.full((16,), _TRASH, jnp.int32) + lax.iota(jnp.int32, 16)
        one_v = jnp.full((16,), 1, jnp.int32)
        c7_v = jnp.full((16,), 7, jnp.int32)
        c127_v = jnp.full((16,), 127, jnp.int32)
        iota_v = lax.iota(jnp.int32, 16)

        def start_seg(g, sbuf, dbuf, sem):
            ebase = sid * _EPT + g * _SEGE
            pltpu.make_async_copy(
                src_hbm.at[pl.ds(ebase, _SEGE)], sbuf, sem).start()
            pltpu.make_async_copy(
                dst_hbm.at[pl.ds(ebase, _SEGE)], dbuf, sem).start()

        def wait_seg(sbuf, dbuf, sem):
            pltpu.make_async_copy(
                src_hbm.at[pl.ds(0, _SEGE)], sbuf, sem).wait()
            pltpu.make_async_copy(
                dst_hbm.at[pl.ds(0, _SEGE)], dbuf, sem).wait()

        def _do_segment(sbuf, dbuf):
            def _cvec(v, off):
                sv = sbuf[pl.ds(v * 16, 16)]
                dv = dbuf[pl.ds(v * 16, 16)]
                keep = ((sv >= slo_v) & (sv < shi_v) &
                        (dv >= dlo_v) & (dv < dhi_v))
                ki = keep.astype(jnp.int32)
                cum = plsc.cumsum(ki)
                off_v = jnp.full((16,), off, jnp.int32)
                pos = (off_v + cum - one_v) * ki + trash_v * (one_v - ki)
                prow = lax.shift_right_logical(pos, c7_v)
                pcol = pos & c127_v
                plsc.store_scatter(list_s, [prow, pcol], sv - slo_v)
                plsc.store_scatter(list_d, [prow, pcol], dv - dlo_v)
                return off + jnp.sum(ki)
            off = lax.fori_loop(0, _SEGE // 16, _cvec, jnp.int32(0))

            zsv = jnp.full((16,), _ZROW, jnp.int32)
            zdv = jnp.zeros((16,), jnp.int32)
            off_v2 = jnp.full((16,), off, jnp.int32)
            for t in range(8):
                ppos = off_v2 + jnp.full((16,), t * 16, jnp.int32) + iota_v
                prow = lax.shift_right_logical(ppos, c7_v)
                pcol = ppos & c127_v
                plsc.store_scatter(list_s, [prow, pcol], zsv)
                plsc.store_scatter(list_d, [prow, pcol], zdv)
            nch = (off + 127) // 128
            nprs = nch // 2
            odd = (nch % 2) == 1

            @pl.when(nch > 0)
            def _():
                start_gather(0, buf0, gsem0)

            def _pair(j, c2):
                i0 = 2 * j
                start_gather(i0 + 1, buf1, gsem1)
                wait_gather(buf0, gsem0)
                pltpu.sync_copy(buf0, acc.at[list_d.at[i0]], add=True)

                @pl.when((j < nprs - 1) | odd)
                def _():
                    start_gather(i0 + 2, buf0, gsem0)

                wait_gather(buf1, gsem1)
                pltpu.sync_copy(buf1, acc.at[list_d.at[i0 + 1]], add=True)
                return c2
            lax.fori_loop(0, nprs, _pair, 0)

            @pl.when(odd)
            def _():
                wait_gather(buf0, gsem0)
                pltpu.sync_copy(buf0, acc.at[list_d.at[2 * nprs]], add=True)

        start_seg(0, src_seg, dst_seg, ssem0)

        def _segpair(g2, c):
            g0 = 2 * g2
            wait_seg(src_seg, dst_seg, ssem0)
            start_seg(g0 + 1, src_seg1, dst_seg1, ssem1)
            _do_segment(src_seg, dst_seg)
            wait_seg(src_seg1, dst_seg1, ssem1)

            @pl.when(g2 < _NSEG // 2 - 1)
            def _():
                start_seg(g0 + 2, src_seg, dst_seg, ssem0)

            _do_segment(src_seg1, dst_seg1)
            return c
        lax.fori_loop(0, _NSEG // 2, _segpair, 0)

    stage_table(cid * _HALF)
    plsc.subcore_barrier()
    run_pass(dlo)
    plsc.subcore_barrier()
    stage_table((1 - cid) * _HALF)
    plsc.subcore_barrier()
    run_pass((1 - cid) * _HALF)
    plsc.subcore_barrier()

    pltpu.sync_copy(acc.at[pl.ds(abase, _SROWS)],
                    out_hbm.at[pl.ds(dlo + abase, _SROWS)])


@functools.cache
def _sc_scatter():
    return pl.kernel(
        _sc_body,
        out_type=jax.ShapeDtypeStruct((_APAD, _D), jnp.float32),
        mesh=plsc.VectorSubcoreMesh(core_axis_name="c", subcore_axis_name="s"),
        scratch_types=[
            pltpu.VMEM((_SEGE,), jnp.int32),
            pltpu.VMEM((_SEGE,), jnp.int32),
            pltpu.VMEM((_SEGE,), jnp.int32),
            pltpu.VMEM((_SEGE,), jnp.int32),
            pltpu.VMEM((_LROWS, _CH), jnp.int32),
            pltpu.VMEM((_LROWS, _CH), jnp.int32),
            pltpu.VMEM((_CH, _D), jnp.float32),
            pltpu.VMEM((_CH, _D), jnp.float32),
            pltpu.VMEM((8, _D), jnp.float32),
            pltpu.VMEM_SHARED((_TPAD, _D), jnp.float32),
            pltpu.VMEM_SHARED((_HALF, _D), jnp.float32),
            pltpu.SemaphoreType.DMA,
            pltpu.SemaphoreType.DMA,
            pltpu.SemaphoreType.DMA,
            pltpu.SemaphoreType.DMA,
        ],
        compiler_params=pltpu.CompilerParams(needs_layout_passes=False),
        name="edge_scatter_add_sc",
    )


_GPB = 10
_AROWS = _GPB * _APG
_FROWS = _GPB * _FPG


def _tc_body(xa, g, xf, wm, ws, wf, out):
    h = jnp.dot(xa[...], ws[...], preferred_element_type=jnp.float32)
    h = h + jnp.dot(g[...], wm[...], preferred_element_type=jnp.float32)
    h = jnp.maximum(h, 0.0)
    f = jnp.maximum(
        jnp.dot(xf[...], wf[...], preferred_element_type=jnp.float32), 0.0)
    hs = jnp.sum(h.reshape(_GPB, _APG, _D), axis=1)
    fs = jnp.sum(f.reshape(_GPB, _FPG, _D), axis=1)
    out[0] = (hs + fs) * (1.0 / (_APG + _FPG))


@functools.partial(jax.jit)
def _tc_finish(x_atom, gp, x_frag, W_msg, W_self, W_frag):
    nb = _B // _GPB
    return pl.pallas_call(
        _tc_body,
        grid=(nb,),
        in_specs=[
            pl.BlockSpec((_AROWS, _D), lambda b: (b, 0)),
            pl.BlockSpec((_AROWS, _D), lambda b: (b, 0)),
            pl.BlockSpec((_FROWS, _D), lambda b: (b, 0)),
            pl.BlockSpec((_D, _D), lambda b: (0, 0)),
            pl.BlockSpec((_D, _D), lambda b: (0, 0)),
            pl.BlockSpec((_D, _D), lambda b: (0, 0)),
        ],
        out_specs=pl.BlockSpec((1, _GPB, _D), lambda b: (b, 0, 0)),
        out_shape=jax.ShapeDtypeStruct((nb, _GPB, _D), jnp.float32),
        name="embed_pool_tc",
    )(x_atom, gp, x_frag, W_msg, W_self, W_frag).reshape(_B, _D)


def kernel(x_atom, x_frag, edge_index, W_msg, W_self, W_frag):
    ei = edge_index.astype(jnp.int32)
    src = ei[0]
    dst = ei[1]
    gp = _sc_scatter()(src, dst, x_atom)
    return _tc_finish(x_atom, gp, x_frag, W_msg, W_self, W_frag)
